# baseline (speedup 1.0000x reference)
import functools

import jax
import jax.numpy as jnp
import numpy as np
from jax import lax
from jax.experimental import pallas as pl
from jax.experimental.pallas import tpu as pltpu
from jax.experimental.pallas import tpu_sc as plsc

VOCAB = 100000
D_MODEL = 768
MAX_LEN = 4096
BASE = 10000
B = 4
S = 4096

N = B * S
NW = 32
POS_PER_W = S // NW
CP = 16
BCP = B * CP
NP = POS_PER_W // CP
GROUPS = D_MODEL // 16
HGROUPS = GROUPS // 2
LANES = 16


def _positional_encoding_np():
    pos = np.arange(MAX_LEN, dtype=np.float32)[:, None]
    i = np.arange(0, D_MODEL, 2, dtype=np.float32)
    div = np.power(float(BASE), i / float(D_MODEL))
    pe = np.zeros((MAX_LEN, D_MODEL), dtype=np.float32)
    pe[:, 0::2] = np.sin(pos / div)
    pe[:, 1::2] = np.cos(pos / div)
    return pe


_PE = _positional_encoding_np()


def _out_row_indices_np():
    w = np.arange(NW)[:, None, None, None]
    jp = np.arange(NP)[None, :, None, None]
    b = np.arange(B)[None, None, :, None]
    i = np.arange(CP)[None, None, None, :]
    rows = b * S + w * POS_PER_W + jp * CP + i
    return rows.reshape(NW, NP, BCP).astype(np.int32)


_OIDX = _out_row_indices_np()

_mesh = plsc.VectorSubcoreMesh(core_axis_name="c", subcore_axis_name="s")


@functools.partial(
    pl.kernel,
    out_type=jax.ShapeDtypeStruct((N, D_MODEL), jnp.float32),
    mesh=_mesh,
    scratch_types=[
        pltpu.VMEM((NP, BCP), jnp.int32),
        pltpu.VMEM((NP, BCP), jnp.int32),
        pltpu.VMEM((2, BCP, D_MODEL), jnp.float32),
        pltpu.VMEM((2, CP, D_MODEL), jnp.float32),
        [pltpu.SemaphoreType.DMA] * 2,
        [pltpu.SemaphoreType.DMA] * 2,
        [pltpu.SemaphoreType.DMA] * 2,
    ],
)
def _embed_sc(idx_hbm, oidx_hbm, table_hbm, pe_hbm, out_hbm,
              idx_v, oidx_v, rbuf, pebuf, gsems, osems, pesems):
    wid = lax.axis_index("s") * 2 + lax.axis_index("c")
    pos0 = wid * POS_PER_W

    pltpu.sync_copy(idx_hbm.at[wid], idx_v)
    pltpu.sync_copy(oidx_hbm.at[wid], oidx_v)

    def start_pe(jp):
        return pltpu.async_copy(
            pe_hbm.at[pl.ds(pos0 + jp * CP, CP)], pebuf.at[jp % 2],
            pesems[jp % 2])

    def start_gather(jp):
        return pltpu.async_copy(
            table_hbm.at[idx_v.at[jp]], rbuf.at[jp % 2], gsems[jp % 2])

    def start_out(jp):
        return pltpu.async_copy(
            rbuf.at[jp % 2], out_hbm.at[oidx_v.at[jp]], osems[jp % 2])

    pe_d = {0: start_pe(0), 1: start_pe(1)}
    g_d = {0: start_gather(0)}
    o_d = {}

    for jp in range(NP):
        q = jp % 2
        if jp + 1 < NP:
            if jp - 1 in o_d:
                o_d[jp - 1].wait()
                o_d[jp - 1] = None
            g_d[jp + 1] = start_gather(jp + 1)
        pe_d[jp].wait()
        g_d[jp].wait()

        def row_body(r, _, q=q, pj=jp % 2):
            for h in range(2):
                cols = [pl.ds((h * HGROUPS + g) * LANES, LANES)
                        for g in range(HGROUPS)]
                pvs = [pebuf[pj, r, c] for c in cols]
                for b in range(B):
                    for g in range(HGROUPS):
                        plsc.addupdate(rbuf.at[q, b * CP + r, cols[g]], pvs[g])
            return 0
        lax.fori_loop(0, CP, row_body, 0)

        o_d[jp] = start_out(jp)
        if jp + 2 < NP:
            pe_d[jp + 2] = start_pe(jp + 2)

    for jp in range(NP - 2, NP):
        if jp in o_d and o_d[jp] is not None:
            o_d[jp].wait()
            o_d[jp] = None


def kernel(x, token_table):
    idx = (x.reshape(B, NW, NP, CP).transpose(1, 2, 0, 3)
           .reshape(NW, NP, BCP).astype(jnp.int32))
    pe = jnp.asarray(_PE)
    oidx = jnp.asarray(_OIDX)
    out = _embed_sc(idx, oidx, token_table, pe)
    return out.reshape(B, S, D_MODEL)


# speedup vs baseline: 1.0388x; 1.0388x over previous
import functools

import jax
import jax.numpy as jnp
import numpy as np
from jax import lax
from jax.experimental import pallas as pl
from jax.experimental.pallas import tpu as pltpu
from jax.experimental.pallas import tpu_sc as plsc

VOCAB = 100000
D_MODEL = 768
MAX_LEN = 4096
BASE = 10000
B = 4
S = 4096

N = B * S
NW = 32
POS_PER_W = S // NW
CP = 16
BCP = B * CP
NP = POS_PER_W // CP
GROUPS = D_MODEL // 16
HGROUPS = GROUPS // 2
LANES = 16


def _positional_encoding_np():
    pos = np.arange(MAX_LEN, dtype=np.float32)[:, None]
    i = np.arange(0, D_MODEL, 2, dtype=np.float32)
    div = np.power(float(BASE), i / float(D_MODEL))
    pe = np.zeros((MAX_LEN, D_MODEL), dtype=np.float32)
    pe[:, 0::2] = np.sin(pos / div)
    pe[:, 1::2] = np.cos(pos / div)
    return pe


_PE = _positional_encoding_np()

_mesh = plsc.VectorSubcoreMesh(core_axis_name="c", subcore_axis_name="s")


@functools.partial(
    pl.kernel,
    out_type=jax.ShapeDtypeStruct((N, D_MODEL), jnp.float32),
    mesh=_mesh,
    scratch_types=[
        pltpu.VMEM((NP, BCP), jnp.int32),
        pltpu.VMEM((2, BCP, D_MODEL), jnp.float32),
        pltpu.VMEM((2, CP, D_MODEL), jnp.float32),
        [pltpu.SemaphoreType.DMA] * 2,
        [[pltpu.SemaphoreType.DMA] * B] * 2,
        [pltpu.SemaphoreType.DMA] * 2,
    ],
)
def _embed_sc(idx_hbm, table_hbm, pe_hbm, out_hbm,
              idx_v, rbuf, pebuf, gsems, osems, pesems):
    wid = lax.axis_index("s") * 2 + lax.axis_index("c")
    pos0 = wid * POS_PER_W

    pltpu.sync_copy(idx_hbm.at[wid], idx_v)

    def start_pe(jp):
        return pltpu.async_copy(
            pe_hbm.at[pl.ds(pos0 + jp * CP, CP)], pebuf.at[jp % 2],
            pesems[jp % 2])

    def start_gather(jp):
        return pltpu.async_copy(
            table_hbm.at[idx_v.at[jp]], rbuf.at[jp % 2], gsems[jp % 2])

    def start_out(jp, b):
        row0 = b * S + pos0 + jp * CP
        return pltpu.async_copy(
            rbuf.at[jp % 2, pl.ds(b * CP, CP)], out_hbm.at[pl.ds(row0, CP)],
            osems[jp % 2][b])

    pe_d = {0: start_pe(0), 1: start_pe(1)}
    g_d = {0: start_gather(0)}
    o_d = {}

    for jp in range(NP):
        q = jp % 2
        if jp + 1 < NP:
            for b in range(B):
                if (jp - 1, b) in o_d:
                    o_d[jp - 1, b].wait()
                    o_d[jp - 1, b] = None
            g_d[jp + 1] = start_gather(jp + 1)
        pe_d[jp].wait()
        g_d[jp].wait()

        def row_body(r, _, q=q, pj=jp % 2):
            for h in range(2):
                cols = [pl.ds((h * HGROUPS + g) * LANES, LANES)
                        for g in range(HGROUPS)]
                pvs = [pebuf[pj, r, c] for c in cols]
                for b in range(B):
                    for g in range(HGROUPS):
                        plsc.addupdate(rbuf.at[q, b * CP + r, cols[g]], pvs[g])
            return 0
        lax.fori_loop(0, CP, row_body, 0)

        for b in range(B):
            o_d[jp, b] = start_out(jp, b)
        if jp + 2 < NP:
            pe_d[jp + 2] = start_pe(jp + 2)

    for jp in range(NP - 2, NP):
        for b in range(B):
            if (jp, b) in o_d and o_d[jp, b] is not None:
                o_d[jp, b].wait()
                o_d[jp, b] = None


def kernel(x, token_table):
    idx = (x.reshape(B, NW, NP, CP).transpose(1, 2, 0, 3)
           .reshape(NW, NP, BCP).astype(jnp.int32))
    pe = jnp.asarray(_PE)
    out = _embed_sc(idx, token_table, pe)
    return out.reshape(B, S, D_MODEL)


# speedup vs baseline: 1.0711x; 1.0311x over previous
import functools

import jax
import jax.numpy as jnp
import numpy as np
from jax import lax
from jax.experimental import pallas as pl
from jax.experimental.pallas import tpu as pltpu
from jax.experimental.pallas import tpu_sc as plsc

VOCAB = 100000
D_MODEL = 768
MAX_LEN = 4096
BASE = 10000
B = 4
S = 4096

N = B * S
NW = 32
POS_PER_W = S // NW
CP = 16
BCP = B * CP
NP = POS_PER_W // CP
GROUPS = D_MODEL // 16
HGROUPS = GROUPS // 2
LANES = 16


def _positional_encoding_np():
    pos = np.arange(MAX_LEN, dtype=np.float32)[:, None]
    i = np.arange(0, D_MODEL, 2, dtype=np.float32)
    div = np.power(float(BASE), i / float(D_MODEL))
    pe = np.zeros((MAX_LEN, D_MODEL), dtype=np.float32)
    pe[:, 0::2] = np.sin(pos / div)
    pe[:, 1::2] = np.cos(pos / div)
    return pe


_PE = _positional_encoding_np()

_mesh = plsc.VectorSubcoreMesh(core_axis_name="c", subcore_axis_name="s")


@functools.partial(
    pl.kernel,
    out_type=jax.ShapeDtypeStruct((N, D_MODEL), jnp.float32),
    mesh=_mesh,
    scratch_types=[
        pltpu.VMEM((NP, BCP), jnp.int32),
        pltpu.VMEM((2, BCP, D_MODEL), jnp.float32),
        pltpu.VMEM((2, CP, D_MODEL), jnp.float32),
        [pltpu.SemaphoreType.DMA] * 2,
        [[pltpu.SemaphoreType.DMA] * B] * 2,
        [pltpu.SemaphoreType.DMA] * 2,
    ],
)
def _embed_sc(idx_hbm, table_hbm, pe_hbm, out_hbm,
              idx_v, rbuf, pebuf, gsems, osems, pesems):
    wid = lax.axis_index("s") * 2 + lax.axis_index("c")
    pos0 = wid * POS_PER_W

    def start_pe(jp):
        return pltpu.async_copy(
            pe_hbm.at[pl.ds(pos0 + jp * CP, CP)], pebuf.at[jp % 2],
            pesems[jp % 2])

    def start_gather(jp):
        return pltpu.async_copy(
            table_hbm.at[idx_v.at[jp]], rbuf.at[jp % 2], gsems[jp % 2])

    def start_out(jp, b):
        row0 = b * S + pos0 + jp * CP
        return pltpu.async_copy(
            rbuf.at[jp % 2, pl.ds(b * CP, CP)], out_hbm.at[pl.ds(row0, CP)],
            osems[jp % 2][b])

    pe_d = {0: start_pe(0), 1: start_pe(1)}
    pltpu.sync_copy(idx_hbm.at[wid, pl.ds(0, 1)], idx_v.at[pl.ds(0, 1)])
    g_d = {0: start_gather(0)}
    pltpu.sync_copy(idx_hbm.at[wid, pl.ds(1, NP - 1)],
                    idx_v.at[pl.ds(1, NP - 1)])
    o_d = {}

    for jp in range(NP):
        q = jp % 2
        if jp + 1 < NP:
            for b in range(B):
                if (jp - 1, b) in o_d:
                    o_d[jp - 1, b].wait()
                    o_d[jp - 1, b] = None
            g_d[jp + 1] = start_gather(jp + 1)
        pe_d[jp].wait()
        g_d[jp].wait()

        def row_body(r, _, q=q, pj=jp % 2):
            for h in range(2):
                cols = [pl.ds((h * HGROUPS + g) * LANES, LANES)
                        for g in range(HGROUPS)]
                pvs = [pebuf[pj, r, c] for c in cols]
                for b in range(B):
                    for g in range(HGROUPS):
                        plsc.addupdate(rbuf.at[q, b * CP + r, cols[g]], pvs[g])
            return 0
        lax.fori_loop(0, CP, row_body, 0)

        for b in range(B):
            o_d[jp, b] = start_out(jp, b)
        if jp + 2 < NP:
            pe_d[jp + 2] = start_pe(jp + 2)

    for jp in range(NP - 2, NP):
        for b in range(B):
            if (jp, b) in o_d and o_d[jp, b] is not None:
                o_d[jp, b].wait()
                o_d[jp, b] = None


def kernel(x, token_table):
    idx = (x.reshape(B, NW, NP, CP).transpose(1, 2, 0, 3)
           .reshape(NW, NP, BCP).astype(jnp.int32))
    pe = jnp.asarray(_PE)
    out = _embed_sc(idx, token_table, pe)
    return out.reshape(B, S, D_MODEL)
